# baseline (speedup 1.0000x reference)
import functools

import jax
import jax.numpy as jnp
from jax import lax
from jax.experimental import pallas as pl
from jax.experimental.pallas import tpu as pltpu
from jax.experimental.pallas import tpu_sc as plsc

NSEG = 512
N = 320000
D = 128
DV = D // 16

NW = 32
ROWS_W = N // NW
C = 80
NCHUNK = ROWS_W // C
G = C // 16


def _sc_body(x_hbm, b_hbm, out_hbm, xbuf, ids, plane, accbuf, sems):
    cid = lax.axis_index("c")
    sid = lax.axis_index("s")
    wid = sid * 2 + cid
    base = wid * ROWS_W

    zero = jnp.zeros((16,), jnp.float32)

    def dma_x(k, slot):
        return pltpu.make_async_copy(
            x_hbm.at[pl.ds(base + k * C, C)], xbuf.at[slot], sems.at[slot]
        )

    idcp = pltpu.make_async_copy(
        b_hbm.at[pl.ds(base, ROWS_W)], ids, sems.at[2]
    )
    idcp.start()
    dma_x(0, 0).start()
    dma_x(1, 1).start()

    def zrow(r, carry):
        prow = plane.at[r]
        for j in range(DV):
            prow[pl.ds(16 * j, 16)] = zero
        return carry

    lax.fori_loop(0, NSEG, zrow, 0)

    for j in range(DV):
        accbuf[pl.ds(16 * j, 16)] = zero

    idcp.wait()
    prev0 = ids[pl.ds(0, 16)][0]

    def groups(k, slot, prev_in):
        xb = xbuf.at[slot]

        def group(g, prev):
            idvec = ids[pl.ds(k * C + g * 16, 16)]
            rbase = g * 16

            def fast(prev):
                for j in range(DV):
                    v = [
                        xb.at[rbase + i][pl.ds(16 * j, 16)] for i in range(16)
                    ]
                    while len(v) > 1:
                        v = [
                            v[t] + v[t + 1] for t in range(0, len(v) - 1, 2)
                        ] + ([v[-1]] if len(v) % 2 else [])
                    accbuf[pl.ds(16 * j, 16)] += v[0]
                return prev

            def slow(prev):
                for i in range(16):
                    seg = idvec[i]
                    changed = seg != prev

                    @pl.when(changed)
                    def _(prev=prev):
                        prow = plane.at[prev]
                        for j in range(DV):
                            prow[pl.ds(16 * j, 16)] = accbuf[
                                pl.ds(16 * j, 16)
                            ]

                    keep = jnp.where(changed, 0.0, 1.0).astype(jnp.float32)
                    xrow = xb.at[rbase + i]
                    for j in range(DV):
                        accbuf[pl.ds(16 * j, 16)] = (
                            accbuf[pl.ds(16 * j, 16)] * keep
                            + xrow[pl.ds(16 * j, 16)]
                        )
                    prev = seg
                return prev

            return lax.cond(idvec[15] == prev, fast, slow, prev)

        return lax.fori_loop(0, G, group, prev_in)

    def pair(p, prev):
        k0 = 2 * p
        dma_x(k0, 0).wait()
        prev = groups(k0, 0, prev)
        dma_x(k0 + 2, 0).start()

        k1 = k0 + 1
        dma_x(k1, 1).wait()
        prev = groups(k1, 1, prev)

        @pl.when(p < (NCHUNK - 1) // 2 - 1)
        def _():
            dma_x(k1 + 2, 1).start()

        return prev

    prev = lax.fori_loop(0, (NCHUNK - 1) // 2, pair, prev0)
    kt = NCHUNK - 1
    dma_x(kt, 0).wait()
    prev = groups(kt, 0, prev)

    prow = plane.at[prev]
    for j in range(DV):
        prow[pl.ds(16 * j, 16)] = accbuf[pl.ds(16 * j, 16)]

    pltpu.sync_copy(plane, out_hbm.at[wid])


def _combine_body(p_ref, o_ref):
    o_ref[...] = jnp.sum(p_ref[...], axis=0)


def kernel(x, batch):
    b32 = batch.astype(jnp.int32)
    sc = pl.kernel(
        _sc_body,
        out_type=jax.ShapeDtypeStruct((NW, NSEG, D), jnp.float32),
        mesh=plsc.VectorSubcoreMesh(core_axis_name="c", subcore_axis_name="s"),
        scratch_types=[
            pltpu.VMEM((2, C, D), jnp.float32),
            pltpu.VMEM((ROWS_W,), jnp.int32),
            pltpu.VMEM((NSEG, D), jnp.float32),
            pltpu.VMEM((D,), jnp.float32),
            pltpu.SemaphoreType.DMA((3,)),
        ],
    )
    partials = sc(x, b32)
    out = pl.pallas_call(
        _combine_body,
        grid=(4,),
        in_specs=[pl.BlockSpec((NW, NSEG // 4, D), lambda i: (0, i, 0))],
        out_specs=pl.BlockSpec((NSEG // 4, D), lambda i: (i, 0)),
        out_shape=jax.ShapeDtypeStruct((NSEG, D), jnp.float32),
    )(partials)
    return out


# speedup vs baseline: 1.1353x; 1.1353x over previous
import functools

import jax
import jax.numpy as jnp
from jax import lax
from jax.experimental import pallas as pl
from jax.experimental.pallas import tpu as pltpu
from jax.experimental.pallas import tpu_sc as plsc

NSEG = 512
N = 320000
D = 128
DV = D // 16

NW = 32
ROWS_W = N // NW
C = 80
NCHUNK = ROWS_W // C
ZR = NSEG // 16


def _sc_body(x_hbm, b2d_hbm, out_hbm, xbuf, ids, zbuf, shared, sems):
    cid = lax.axis_index("c")
    sid = lax.axis_index("s")
    wid = sid * 2 + cid
    base = wid * ROWS_W

    def dma_x(k, slot):
        return pltpu.make_async_copy(
            x_hbm.at[pl.ds(base + k * C, C)], xbuf.at[slot], sems.at[slot]
        )

    idcp = pltpu.make_async_copy(b2d_hbm.at[wid], ids, sems.at[2])
    idcp.start()

    zero = jnp.zeros((16,), jnp.float32)

    def zrow(r, carry):
        row = zbuf.at[r]
        for j in range(DV):
            row[pl.ds(16 * j, 16)] = zero
        return carry

    lax.fori_loop(0, ZR, zrow, 0)
    pltpu.sync_copy(zbuf, shared.at[pl.ds(sid * ZR, ZR)])
    plsc.subcore_barrier()

    dma_x(0, 0).start()
    dma_x(1, 1).start()
    idcp.wait()

    def scat(k, slot):
        pltpu.sync_copy(xbuf.at[slot], shared.at[ids.at[k]], add=True)

    def pair(p, carry):
        k0 = 2 * p
        dma_x(k0, 0).wait()
        scat(k0, 0)
        dma_x(k0 + 2, 0).start()

        k1 = k0 + 1
        dma_x(k1, 1).wait()
        scat(k1, 1)

        @pl.when(p < (NCHUNK - 1) // 2 - 1)
        def _():
            dma_x(k1 + 2, 1).start()

        return carry

    lax.fori_loop(0, (NCHUNK - 1) // 2, pair, 0)
    kt = NCHUNK - 1
    dma_x(kt, 0).wait()
    scat(kt, 0)

    plsc.subcore_barrier()
    pltpu.sync_copy(
        shared.at[pl.ds(sid * ZR, ZR)],
        out_hbm.at[cid].at[pl.ds(sid * ZR, ZR)],
    )


def _combine_body(p_ref, o_ref):
    o_ref[...] = p_ref[0] + p_ref[1]


def kernel(x, batch):
    b2d = batch.astype(jnp.int32).reshape(NW, NCHUNK, C)
    sc = pl.kernel(
        _sc_body,
        out_type=jax.ShapeDtypeStruct((2, NSEG, D), jnp.float32),
        mesh=plsc.VectorSubcoreMesh(core_axis_name="c", subcore_axis_name="s"),
        scratch_types=[
            pltpu.VMEM((2, C, D), jnp.float32),
            pltpu.VMEM((NCHUNK, C), jnp.int32),
            pltpu.VMEM((ZR, D), jnp.float32),
            pltpu.VMEM_SHARED((NSEG, D), jnp.float32),
            pltpu.SemaphoreType.DMA((3,)),
        ],
    )
    partials = sc(x, b2d)
    out = pl.pallas_call(
        _combine_body,
        out_shape=jax.ShapeDtypeStruct((NSEG, D), jnp.float32),
    )(partials)
    return out
